# Initial kernel scaffold; baseline (speedup 1.0000x reference)
import jax
import jax.numpy as jnp
from jax.experimental import pallas as pl


def kernel(in_flow):
    raise NotImplementedError("write your pallas kernel here")


# speedup vs baseline: 1.0132x; 1.0132x over previous
import jax
import jax.numpy as jnp
from jax.experimental import pallas as pl

_LANES = 128


def _tag_kernel(tag_ref, load_ref):
    rows = tag_ref.shape[0]
    row = jax.lax.broadcasted_iota(jnp.int32, (rows, _LANES), 0)
    col = jax.lax.broadcasted_iota(jnp.int32, (rows, _LANES), 1)
    tag_ref[...] = row * _LANES + col
    load_ref[...] = jnp.full((1, 1), rows * _LANES, jnp.int32)


def kernel(in_flow):
    n = in_flow.shape[0]
    rows = n // _LANES
    tag2d, load = pl.pallas_call(
        _tag_kernel,
        out_shape=(
            jax.ShapeDtypeStruct((rows, _LANES), jnp.int32),
            jax.ShapeDtypeStruct((1, 1), jnp.int32),
        ),
    )()
    return (in_flow, tag2d.reshape(n, 1), load.reshape(()))
